# baseline (speedup 1.0000x reference)
import functools

import numpy as np
import jax
import jax.numpy as jnp
from jax import lax
from jax.experimental import pallas as pl
from jax.experimental.pallas import tpu as pltpu
from jax.experimental.pallas import tpu_sc as plsc

_S = 64.0
_M = 0.5
_STD = 0.0125
_EPS = 1e-6

_NW = 32
_RPW = 32


def _margin_cs(n: int):
    m = _M + _STD * jax.random.normal(jax.random.key(42), (n,), dtype=jnp.float32)
    return jnp.cos(m), jnp.sin(m)


def _sqrt16(v):
    i = plsc.bitcast(v, jnp.int32)
    t = plsc.bitcast((i >> 1) + jnp.int32(0x1FBD1DF5), jnp.float32)
    for _ in range(3):
        t = 0.5 * (t + v / t)
    return t


def _sc_body(ct_hbm, lbl_hbm, cm_hbm, sm_hbm, fix_hbm,
             lbl_v, blk_v, cm_v, sm_v, fix_v, sem):
    wid = lax.axis_index("s") * 2 + lax.axis_index("c")
    base = wid * _RPW
    pltpu.sync_copy(lbl_hbm.at[pl.ds(base, _RPW)], lbl_v)
    pltpu.sync_copy(cm_hbm.at[pl.ds(base, _RPW)], cm_v)
    pltpu.sync_copy(sm_hbm.at[pl.ds(base, _RPW)], sm_v)
    lane_ids = lax.iota(jnp.int32, 16)
    chunks = [lbl_v[pl.ds(c * 16, 16)] for c in range(_RPW // 16)]
    copies = []
    for i in range(_RPW):
        l = jnp.max(jnp.where(lane_ids == (i % 16), chunks[i // 16], 0))
        c0 = pl.multiple_of((l >> 7) << 7, 128)
        r0 = pl.multiple_of(base + (i & ~7), 8)
        copies.append(pltpu.async_copy(
            ct_hbm.at[pl.ds(r0, 8), pl.ds(c0, 128)], blk_v.at[i], sem))
    for cp in copies:
        cp.wait()
    for c in range(_RPW // 16):
        ii = c * 16 + lax.iota(jnp.int32, 16)
        lbl16 = lbl_v[pl.ds(c * 16, 16)]
        x = plsc.load_gather(blk_v, [ii, ii & 7, lbl16 & 127])
        y = jnp.clip(x, -1.0 + _EPS, 1.0 - _EPS)
        s = _sqrt16(1.0 - y * y)
        f = (y * cm_v[pl.ds(c * 16, 16)] - s * sm_v[pl.ds(c * 16, 16)]) * _S
        fix_v[pl.ds(c * 16, 16)] = f
    pltpu.sync_copy(fix_v, fix_hbm.at[pl.ds(base, _RPW)])


def _sc_fix(cos_theta, label, cm, sm):
    b = label.shape[0]
    return pl.kernel(
        _sc_body,
        out_type=jax.ShapeDtypeStruct((b,), jnp.float32),
        mesh=plsc.VectorSubcoreMesh(core_axis_name="c", subcore_axis_name="s"),
        compiler_params=pltpu.CompilerParams(needs_layout_passes=False),
        scratch_types=[
            pltpu.VMEM((_RPW,), jnp.int32),
            pltpu.VMEM((_RPW, 8, 128), jnp.float32),
            pltpu.VMEM((_RPW,), jnp.float32),
            pltpu.VMEM((_RPW,), jnp.float32),
            pltpu.VMEM((_RPW,), jnp.float32),
            pltpu.SemaphoreType.DMA,
        ],
    )(cos_theta, label, cm, sm)


_W = 4352
_NCK = 23


def _sc_dense_body(ct_hbm, lbl_hbm, cm_hbm, sm_hbm, out_hbm,
                   lbl_v, cm_v, sm_v, b0, b1, b2,
                   si0, si1, si2, so0, so1, so2):
    bufs = (b0, b1, b2)
    sin = (si0, si1, si2)
    sout = (so0, so1, so2)
    wid = lax.axis_index("s") * 2 + lax.axis_index("c")
    base = wid * _RPW
    pltpu.sync_copy(lbl_hbm.at[pl.ds(base, _RPW)], lbl_v)
    pltpu.sync_copy(cm_hbm.at[pl.ds(base, _RPW)], cm_v)
    pltpu.sync_copy(sm_hbm.at[pl.ds(base, _RPW)], sm_v)
    lane_ids = lax.iota(jnp.int32, 16)
    chunks = [lbl_v[pl.ds(c * 16, 16)] for c in range(_RPW // 16)]
    lbls = [jnp.max(jnp.where(lane_ids == (i % 16), chunks[i // 16], 0))
            for i in range(_RPW)]

    def _compute(buf):
        def step(k, _):
            o = pl.multiple_of(k * 16, 16)
            for r in range(8):
                v = buf[r, pl.ds(o, 16)]
                buf[r, pl.ds(o, 16)] = jnp.clip(
                    v, -1.0 + _EPS, 1.0 - _EPS) * _S
            return 0
        lax.fori_loop(0, _W // 16, step, 0)

    def _patch(buf, g, c0):
        for r in range(8):
            l = lbls[g * 8 + r]

            @pl.when((l >= c0) & (l < c0 + _W))
            def _():
                posv = jnp.broadcast_to(l - c0, (16,))
                rv = jnp.full((16,), r, jnp.int32)
                iv = jnp.full((16,), g * 8 + r, jnp.int32)
                y = plsc.load_gather(buf, [rv, posv]) * (1.0 / _S)
                s = _sqrt16(1.0 - y * y)
                f = (y * plsc.load_gather(cm_v, [iv])
                     - s * plsc.load_gather(sm_v, [iv])) * _S
                plsc.store_scatter(buf, [rv, posv], f, mask=lane_ids == 0)

    for g in range(4):
        r0 = pl.multiple_of(base + g * 8, 8)

        def _in_slice(cc):
            c0 = pl.multiple_of(cc * _W, 128)
            return ct_hbm.at[pl.ds(r0, 8), pl.ds(c0, _W)]

        def _out_slice(cc):
            c0 = pl.multiple_of(cc * _W, 128)
            return out_hbm.at[pl.ds(r0, 8), pl.ds(c0, _W)]

        pltpu.async_copy(_in_slice(0), bufs[0], sin[0])

        def chunk_body(cc, _):
            c0 = pl.multiple_of(cc * _W, 128)
            for b in range(3):
                @pl.when(cc % 3 == b)
                def _():
                    pltpu.make_async_copy(_in_slice(cc), bufs[b], sin[b]).wait()
                    _compute(bufs[b])
                    _patch(bufs[b], g, c0)
                    pltpu.async_copy(bufs[b], _out_slice(cc), sout[b])
            for b in range(3):
                @pl.when((cc + 1 < _NCK) & ((cc + 1) % 3 == b))
                def _():
                    @pl.when(cc >= 2)
                    def _():
                        pltpu.make_async_copy(
                            bufs[b], _out_slice(cc), sout[b]).wait()
                    pltpu.async_copy(_in_slice(cc + 1), bufs[b], sin[b])
            return 0

        lax.fori_loop(0, _NCK, chunk_body, 0)
        for b in range(3):
            last = _NCK - 1 - ((_NCK - 1 - b) % 3)
            if 0 <= last < _NCK:
                pltpu.make_async_copy(bufs[b], _out_slice(last), sout[b]).wait()


def _sc_dense(cos_theta, label, cm, sm):
    b, c = cos_theta.shape
    return pl.kernel(
        _sc_dense_body,
        out_type=jax.ShapeDtypeStruct((b, c), jnp.float32),
        mesh=plsc.VectorSubcoreMesh(core_axis_name="c", subcore_axis_name="s"),
        compiler_params=pltpu.CompilerParams(needs_layout_passes=False),
        scratch_types=[
            pltpu.VMEM((_RPW,), jnp.int32),
            pltpu.VMEM((_RPW,), jnp.float32),
            pltpu.VMEM((_RPW,), jnp.float32),
            pltpu.VMEM((8, _W), jnp.float32),
            pltpu.VMEM((8, _W), jnp.float32),
            pltpu.VMEM((8, _W), jnp.float32),
            pltpu.SemaphoreType.DMA,
            pltpu.SemaphoreType.DMA,
            pltpu.SemaphoreType.DMA,
            pltpu.SemaphoreType.DMA,
            pltpu.SemaphoreType.DMA,
            pltpu.SemaphoreType.DMA,
        ],
    )(cos_theta, label, cm, sm)


@jax.jit
def _arcface_sc(cos_theta, label):
    cm, sm = _margin_cs(label.shape[0])
    return _sc_dense(cos_theta, label, cm, sm)


def _tc_body(x_ref, lbl_ref, fix_ref, o_ref):
    x = x_ref[...]
    y = jnp.clip(x, -1.0 + _EPS, 1.0 - _EPS)
    cols = lax.broadcasted_iota(jnp.int32, x.shape, 1)
    mask = cols == lbl_ref[...]
    o_ref[...] = jnp.where(mask, fix_ref[...], y * _S)


@functools.partial(jax.jit, static_argnames=("rb",))
def _arcface(cos_theta, label, rb=16):
    b, c = cos_theta.shape
    cm, sm = _margin_cs(b)
    fix = _sc_fix(cos_theta, label, cm, sm)
    return pl.pallas_call(
        _tc_body,
        grid=(pl.cdiv(b, rb),),
        in_specs=[
            pl.BlockSpec((rb, c), lambda j: (j, 0)),
            pl.BlockSpec((rb, 1), lambda j: (j, 0)),
            pl.BlockSpec((rb, 1), lambda j: (j, 0)),
        ],
        out_specs=pl.BlockSpec((rb, c), lambda j: (j, 0)),
        out_shape=jax.ShapeDtypeStruct((b, c), jnp.float32),
    )(cos_theta, label.reshape(b, 1), fix.reshape(b, 1))


def kernel(cos_theta, label):
    b, c = cos_theta.shape
    if b % (_NW * _RPW // 32) == 0 and b % 32 == 0 and c == 100000:
        return _arcface_sc(cos_theta, label)
    return _arcface(cos_theta, label)


# speedup vs baseline: 1.3948x; 1.3948x over previous
import functools

import numpy as np
import jax
import jax.numpy as jnp
from jax import lax
from jax.experimental import pallas as pl
from jax.experimental.pallas import tpu as pltpu
from jax.experimental.pallas import tpu_sc as plsc

_S = 64.0
_M = 0.5
_STD = 0.0125
_EPS = 1e-6

_NW = 32
_RPW = 32


def _margin_cs(n: int):
    m = _M + _STD * jax.random.normal(jax.random.key(42), (n,), dtype=jnp.float32)
    return jnp.cos(m), jnp.sin(m)


def _sqrt16(v):
    i = plsc.bitcast(v, jnp.int32)
    t = plsc.bitcast((i >> 1) + jnp.int32(0x1FBD1DF5), jnp.float32)
    for _ in range(3):
        t = 0.5 * (t + v / t)
    return t


def _sc_body(ct_hbm, lbl_hbm, cm_hbm, sm_hbm, fix_hbm,
             lbl_v, blk_v, cm_v, sm_v, fix_v, sem):
    wid = lax.axis_index("s") * 2 + lax.axis_index("c")
    base = wid * _RPW
    pltpu.sync_copy(lbl_hbm.at[pl.ds(base, _RPW)], lbl_v)
    pltpu.sync_copy(cm_hbm.at[pl.ds(base, _RPW)], cm_v)
    pltpu.sync_copy(sm_hbm.at[pl.ds(base, _RPW)], sm_v)
    lane_ids = lax.iota(jnp.int32, 16)
    chunks = [lbl_v[pl.ds(c * 16, 16)] for c in range(_RPW // 16)]
    copies = []
    for i in range(_RPW):
        l = jnp.max(jnp.where(lane_ids == (i % 16), chunks[i // 16], 0))
        c0 = pl.multiple_of((l >> 7) << 7, 128)
        r0 = pl.multiple_of(base + (i & ~7), 8)
        copies.append(pltpu.async_copy(
            ct_hbm.at[pl.ds(r0, 8), pl.ds(c0, 128)], blk_v.at[i], sem))
    for cp in copies:
        cp.wait()
    for c in range(_RPW // 16):
        ii = c * 16 + lax.iota(jnp.int32, 16)
        lbl16 = lbl_v[pl.ds(c * 16, 16)]
        x = plsc.load_gather(blk_v, [ii, ii & 7, lbl16 & 127])
        y = jnp.clip(x, -1.0 + _EPS, 1.0 - _EPS)
        s = _sqrt16(1.0 - y * y)
        f = (y * cm_v[pl.ds(c * 16, 16)] - s * sm_v[pl.ds(c * 16, 16)]) * _S
        fix_v[pl.ds(c * 16, 16)] = f
    pltpu.sync_copy(fix_v, fix_hbm.at[pl.ds(base, _RPW)])


def _sc_fix(cos_theta, label, cm, sm):
    b = label.shape[0]
    return pl.kernel(
        _sc_body,
        out_type=jax.ShapeDtypeStruct((b,), jnp.float32),
        mesh=plsc.VectorSubcoreMesh(core_axis_name="c", subcore_axis_name="s"),
        compiler_params=pltpu.CompilerParams(needs_layout_passes=False),
        scratch_types=[
            pltpu.VMEM((_RPW,), jnp.int32),
            pltpu.VMEM((_RPW, 8, 128), jnp.float32),
            pltpu.VMEM((_RPW,), jnp.float32),
            pltpu.VMEM((_RPW,), jnp.float32),
            pltpu.VMEM((_RPW,), jnp.float32),
            pltpu.SemaphoreType.DMA,
        ],
    )(cos_theta, label, cm, sm)


def _tc_body(x_ref, lbl_ref, fix_ref, o_ref):
    x = x_ref[...]
    y = jnp.clip(x, -1.0 + _EPS, 1.0 - _EPS)
    cols = lax.broadcasted_iota(jnp.int32, x.shape, 1)
    mask = cols == lbl_ref[...]
    o_ref[...] = jnp.where(mask, fix_ref[...], y * _S)


@functools.partial(jax.jit, static_argnames=("rb",))
def _arcface(cos_theta, label, rb=16):
    b, c = cos_theta.shape
    cm, sm = _margin_cs(b)
    fix = _sc_fix(cos_theta, label, cm, sm)
    return pl.pallas_call(
        _tc_body,
        grid=(pl.cdiv(b, rb),),
        in_specs=[
            pl.BlockSpec((rb, c), lambda j: (j, 0)),
            pl.BlockSpec((rb, 1), lambda j: (j, 0)),
            pl.BlockSpec((rb, 1), lambda j: (j, 0)),
        ],
        out_specs=pl.BlockSpec((rb, c), lambda j: (j, 0)),
        out_shape=jax.ShapeDtypeStruct((b, c), jnp.float32),
    )(cos_theta, label.reshape(b, 1), fix.reshape(b, 1))


def kernel(cos_theta, label):
    return _arcface(cos_theta, label)


# speedup vs baseline: 1.3956x; 1.0006x over previous
import functools

import jax
import jax.numpy as jnp
from jax import lax
from jax.experimental import pallas as pl
from jax.experimental.pallas import tpu as pltpu
from jax.experimental.pallas import tpu_sc as plsc

_S = 64.0
_M = 0.5
_STD = 0.0125
_EPS = 1e-6

_RPW = 32


def _margin_cs(n: int):
    m = _M + _STD * jax.random.normal(jax.random.key(42), (n,), dtype=jnp.float32)
    return jnp.cos(m), jnp.sin(m)


def _sqrt16(v):
    i = plsc.bitcast(v, jnp.int32)
    t = plsc.bitcast((i >> 1) + jnp.int32(0x1FBD1DF5), jnp.float32)
    for _ in range(3):
        t = 0.5 * (t + v / t)
    return t


def _sc_body(ct_hbm, lbl_hbm, cm_hbm, sm_hbm, fix_hbm,
             lbl_v, blk_v, cm_v, sm_v, fix_v, sem):
    wid = lax.axis_index("s") * 2 + lax.axis_index("c")
    base = wid * _RPW
    pltpu.sync_copy(lbl_hbm.at[pl.ds(base, _RPW)], lbl_v)
    pltpu.sync_copy(cm_hbm.at[pl.ds(base, _RPW)], cm_v)
    pltpu.sync_copy(sm_hbm.at[pl.ds(base, _RPW)], sm_v)
    lane_ids = lax.iota(jnp.int32, 16)
    chunks = [lbl_v[pl.ds(c * 16, 16)] for c in range(_RPW // 16)]
    copies = []
    for i in range(_RPW):
        l = jnp.max(jnp.where(lane_ids == (i % 16), chunks[i // 16], 0))
        c0 = pl.multiple_of((l >> 7) << 7, 128)
        r0 = pl.multiple_of(base + (i & ~7), 8)
        copies.append(pltpu.async_copy(
            ct_hbm.at[pl.ds(r0, 8), pl.ds(c0, 128)], blk_v.at[i], sem))
    for cp in copies:
        cp.wait()
    for c in range(_RPW // 16):
        ii = c * 16 + lax.iota(jnp.int32, 16)
        lbl16 = lbl_v[pl.ds(c * 16, 16)]
        x = plsc.load_gather(blk_v, [ii, ii & 7, lbl16 & 127])
        y = jnp.clip(x, -1.0 + _EPS, 1.0 - _EPS)
        s = _sqrt16(1.0 - y * y)
        f = (y * cm_v[pl.ds(c * 16, 16)] - s * sm_v[pl.ds(c * 16, 16)]) * _S
        fix_v[pl.ds(c * 16, 16)] = f
    pltpu.sync_copy(fix_v, fix_hbm.at[pl.ds(base, _RPW)])


def _sc_fix(cos_theta, label, cm, sm):
    b = label.shape[0]
    return pl.kernel(
        _sc_body,
        out_type=jax.ShapeDtypeStruct((b,), jnp.float32),
        mesh=plsc.VectorSubcoreMesh(core_axis_name="c", subcore_axis_name="s"),
        compiler_params=pltpu.CompilerParams(needs_layout_passes=False),
        scratch_types=[
            pltpu.VMEM((_RPW,), jnp.int32),
            pltpu.VMEM((_RPW, 8, 128), jnp.float32),
            pltpu.VMEM((_RPW,), jnp.float32),
            pltpu.VMEM((_RPW,), jnp.float32),
            pltpu.VMEM((_RPW,), jnp.float32),
            pltpu.SemaphoreType.DMA,
        ],
    )(cos_theta, label, cm, sm)


def _tc_body(x_ref, lbl_ref, fix_ref, o_ref):
    x = x_ref[...]
    y = jnp.clip(x, -1.0 + _EPS, 1.0 - _EPS)
    cols = lax.broadcasted_iota(jnp.int32, x.shape, 1)
    mask = cols == lbl_ref[...]
    o_ref[...] = jnp.where(mask, fix_ref[...], y * _S)


@functools.partial(jax.jit, static_argnames=("rb",))
def _arcface(cos_theta, label, rb=16):
    b, c = cos_theta.shape
    cm, sm = _margin_cs(b)
    fix = _sc_fix(cos_theta, label, cm, sm)
    return pl.pallas_call(
        _tc_body,
        grid=(pl.cdiv(b, rb),),
        in_specs=[
            pl.BlockSpec((rb, c), lambda j: (j, 0)),
            pl.BlockSpec((rb, 1), lambda j: (j, 0)),
            pl.BlockSpec((rb, 1), lambda j: (j, 0)),
        ],
        out_specs=pl.BlockSpec((rb, c), lambda j: (j, 0)),
        out_shape=jax.ShapeDtypeStruct((b, c), jnp.float32),
    )(cos_theta, label.reshape(b, 1), fix.reshape(b, 1))


def kernel(cos_theta, label):
    return _arcface(cos_theta, label)
